# baseline (speedup 1.0000x reference)
import functools

import jax
import jax.numpy as jnp
from jax import lax
from jax.experimental import pallas as pl
from jax.experimental.pallas import tpu as pltpu
from jax.experimental.pallas import tpu_sc as plsc

B = 16384
N_CAT = 26
N_CONT = 16
N_TOK = N_CAT + N_CONT
CARD = 100000
DIM = 32

NC = 2
NS = 16
NW = NC * NS

B_W = B // NW
CAT_ROWS_W = B_W * N_CAT
R_CAT = 512
N_CAT_CHUNKS = CAT_ROWS_W // R_CAT
CB = 32
R_CONT = CB * N_CONT
N_CONT_CHUNKS = B_W // CB
IDXW = 128


def _sc_body(gidx_hbm, ocat_hbm, ocont_hbm, xc_hbm, w_hbm, b_hbm, table_hbm,
             out_hbm,
             idx_v, oidx_v, rows_v, coidx_v, crow_v, xv, wv, bv,
             gsem, ssem, csem):
    wid = lax.axis_index("s") * NC + lax.axis_index("c")

    pltpu.sync_copy(w_hbm, wv)
    pltpu.sync_copy(b_hbm, bv)

    def cat_chunk(i, carry):
        g = wid * N_CAT_CHUNKS + i
        r0 = g * R_CAT
        pltpu.sync_copy(gidx_hbm.at[pl.ds(r0, R_CAT)], idx_v)
        pltpu.sync_copy(ocat_hbm.at[pl.ds(r0, R_CAT)], oidx_v)
        pltpu.async_copy(table_hbm.at[idx_v], rows_v, gsem).wait()
        pltpu.async_copy(rows_v, out_hbm.at[oidx_v], ssem).wait()
        return carry

    lax.fori_loop(0, N_CAT_CHUNKS, cat_chunk, 0)

    def cont_chunk(i, carry):
        g = wid * N_CONT_CHUNKS + i
        pltpu.sync_copy(xc_hbm.at[pl.ds(g * R_CONT, R_CONT)], xv)
        pltpu.sync_copy(ocont_hbm.at[pl.ds(g * R_CONT, R_CONT)], coidx_v)
        for f in range(N_CONT):
            w0 = wv[f, pl.ds(0, 16)]
            w1 = wv[f, pl.ds(16, 16)]
            b0 = bv[f, pl.ds(0, 16)]
            b1 = bv[f, pl.ds(16, 16)]

            def jbody(j, c, f=f, w0=w0, w1=w1, b0=b0, b1=b1):
                xrow = xv[pl.ds(j * N_CONT, N_CONT)]
                xs = xrow[f]
                r = j * N_CONT + f
                crow_v[r, pl.ds(0, 16)] = xs * w0 + b0
                crow_v[r, pl.ds(16, 16)] = xs * w1 + b1
                return c

            lax.fori_loop(0, CB, jbody, 0)
        pltpu.async_copy(crow_v, out_hbm.at[coidx_v], csem).wait()
        return carry

    lax.fori_loop(0, N_CONT_CHUNKS, cont_chunk, 0)


@jax.jit
def kernel(x_cat, x_cont, cat_table, cont_W, cont_b):
    offsets = jnp.arange(N_CAT, dtype=jnp.int32) * CARD
    gidx = (x_cat.astype(jnp.int32) + offsets[None, :]).reshape(-1)
    brow = jnp.arange(B, dtype=jnp.int32) * N_TOK
    ocat = (brow[:, None] + jnp.arange(N_CAT, dtype=jnp.int32)[None, :])
    ocont = (brow[:, None] + N_CAT
             + jnp.arange(N_CONT, dtype=jnp.int32)[None, :])

    ocat = ocat.reshape(-1)
    ocont = ocont.reshape(-1)
    xc = x_cont.reshape(-1)

    mesh = plsc.VectorSubcoreMesh(core_axis_name="c", subcore_axis_name="s",
                                  num_cores=NC, num_subcores=NS)
    out = pl.kernel(
        _sc_body,
        out_type=jax.ShapeDtypeStruct((B * N_TOK, DIM), jnp.float32),
        mesh=mesh,
        scratch_types=[
            pltpu.VMEM((R_CAT,), jnp.int32),
            pltpu.VMEM((R_CAT,), jnp.int32),
            pltpu.VMEM((R_CAT, DIM), jnp.float32),
            pltpu.VMEM((R_CONT,), jnp.int32),
            pltpu.VMEM((R_CONT, DIM), jnp.float32),
            pltpu.VMEM((R_CONT,), jnp.float32),
            pltpu.VMEM((N_CONT, DIM), jnp.float32),
            pltpu.VMEM((N_CONT, DIM), jnp.float32),
            pltpu.SemaphoreType.DMA,
            pltpu.SemaphoreType.DMA,
            pltpu.SemaphoreType.DMA,
        ],
        compiler_params=pltpu.CompilerParams(use_tc_tiling_on_sc=False),
    )(gidx, ocat, ocont, xc, cont_W, cont_b, cat_table)
    return out.reshape(B, N_TOK, DIM)
